# Initial kernel scaffold; baseline (speedup 1.0000x reference)
import jax
import jax.numpy as jnp
from jax.experimental import pallas as pl


def kernel(x, e, edge_index, sub_edge_ids, Wn, bn, We, be, A, B, C, D, Ee, Wp, bp):
    raise NotImplementedError("write your pallas kernel here")


# speedup vs baseline: 1.5666x; 1.5666x over previous
import functools

import jax
import jax.numpy as jnp
from jax import lax
from jax.experimental import pallas as pl
from jax.experimental.pallas import tpu as pltpu
from jax.experimental.pallas import tpu_sc as plsc

N_NODES = 10000
N_EDGES = 160000
D_NODE = 128
D_EDGE = 16
H = 128
HH = 64
L = 3
P = 32768

NS = 16
EPT = N_EDGES // NS
NB = 40
NBLK = EPT // NB
NPT = 624
ZB = 104
NZB = NPT // ZB
NTAIL = N_NODES - NPT * NS

ER = N_EDGES // 128
NR = 80
NPAD = NR * 128

_mesh = plsc.VectorSubcoreMesh(core_axis_name="c", subcore_axis_name="s")


def _f32(*shape):
    return jax.ShapeDtypeStruct(shape, jnp.float32)


def _enc_h_body(x_ref, wn_ref, bn_ref, h_ref):
    h_ref[...] = jnp.dot(x_ref[...], wn_ref[...],
                         preferred_element_type=jnp.float32) + bn_ref[...]


def _enc_h(x, Wn, bn):
    bs = 2000
    return pl.pallas_call(
        _enc_h_body,
        grid=(N_NODES // bs,),
        in_specs=[
            pl.BlockSpec((bs, D_NODE), lambda i: (i, 0)),
            pl.BlockSpec((D_NODE, H), lambda i: (0, 0)),
            pl.BlockSpec((1, H), lambda i: (0, 0)),
        ],
        out_specs=pl.BlockSpec((bs, H), lambda i: (i, 0)),
        out_shape=_f32(N_NODES, H),
    )(x, Wn, bn.reshape(1, H))


def _enc_e_body(e_ref, we_ref, be_ref, lo_ref, hi_ref):
    r = jnp.dot(e_ref[...], we_ref[...],
                preferred_element_type=jnp.float32) + be_ref[...]
    lo_ref[...] = r[:, :HH]
    hi_ref[...] = r[:, HH:]


def _enc_e(e, We, be):
    bs = 4000
    return pl.pallas_call(
        _enc_e_body,
        grid=(N_EDGES // bs,),
        in_specs=[
            pl.BlockSpec((bs, D_EDGE), lambda i: (i, 0)),
            pl.BlockSpec((D_EDGE, H), lambda i: (0, 0)),
            pl.BlockSpec((1, H), lambda i: (0, 0)),
        ],
        out_specs=[pl.BlockSpec((bs, HH), lambda i: (i, 0))] * 2,
        out_shape=[_f32(N_EDGES, HH)] * 2,
    )(e, We, be.reshape(1, H))


def _proj_body(h_ref, d_ref, ee_ref, b_ref, db0_ref, db1_ref, he_ref):
    h = h_ref[...]
    rd = jnp.dot(h, d_ref[...], preferred_element_type=jnp.float32)
    rb = jnp.dot(h, b_ref[...], preferred_element_type=jnp.float32)
    he_ref[...] = jnp.dot(h, ee_ref[...], preferred_element_type=jnp.float32)
    db0_ref[...] = jnp.concatenate([rd[:, :HH], rb[:, :HH]], axis=1)
    db1_ref[...] = jnp.concatenate([rd[:, HH:], rb[:, HH:]], axis=1)


def _proj(h, Dl, Eel, Bl):
    bs = 2000
    return pl.pallas_call(
        _proj_body,
        grid=(N_NODES // bs,),
        in_specs=[
            pl.BlockSpec((bs, H), lambda i: (i, 0)),
            pl.BlockSpec((H, H), lambda i: (0, 0)),
            pl.BlockSpec((H, H), lambda i: (0, 0)),
            pl.BlockSpec((H, H), lambda i: (0, 0)),
        ],
        out_specs=[pl.BlockSpec((bs, H), lambda i: (i, 0))] * 3,
        out_shape=[_f32(N_NODES, H)] * 3,
    )(h, Dl, Eel, Bl)


def _efc_body(lo_ref, hi_ref, c_ref, o0_ref, o1_ref):
    c = c_ref[...]
    r = (jnp.dot(lo_ref[...], c[:HH, :], preferred_element_type=jnp.float32)
         + jnp.dot(hi_ref[...], c[HH:, :], preferred_element_type=jnp.float32))
    o0_ref[...] = r[:, :HH]
    o1_ref[...] = r[:, HH:]


def _efc(ef0, ef1, Cl):
    bs = 4000
    return pl.pallas_call(
        _efc_body,
        grid=(N_EDGES // bs,),
        in_specs=[
            pl.BlockSpec((bs, HH), lambda i: (i, 0)),
            pl.BlockSpec((bs, HH), lambda i: (i, 0)),
            pl.BlockSpec((H, H), lambda i: (0, 0)),
        ],
        out_specs=[pl.BlockSpec((bs, HH), lambda i: (i, 0))] * 2,
        out_shape=[_f32(N_EDGES, HH)] * 2,
    )(ef0, ef1, Cl)


def _update_body(h_ref, a_ref, x0_ref, x1_ref, o_ref):
    h = h_ref[...]
    a0 = x0_ref[...]
    a1 = x1_ref[...]
    agg = jnp.concatenate([a0[:, :HH], a1[:, :HH]], axis=1)
    den = jnp.concatenate([a0[:, HH:], a1[:, HH:]], axis=1) + 1e-6
    hn = jnp.dot(h, a_ref[...], preferred_element_type=jnp.float32) + agg / den
    o_ref[...] = h + jnp.maximum(hn, 0.0)


def _update(h, Al, acc0, acc1):
    bs = 2000
    return pl.pallas_call(
        _update_body,
        grid=(N_NODES // bs,),
        in_specs=[
            pl.BlockSpec((bs, H), lambda i: (i, 0)),
            pl.BlockSpec((H, H), lambda i: (0, 0)),
            pl.BlockSpec((bs, H), lambda i: (i, 0)),
            pl.BlockSpec((bs, H), lambda i: (i, 0)),
        ],
        out_specs=pl.BlockSpec((bs, H), lambda i: (i, 0)),
        out_shape=_f32(N_NODES, H),
    )(h, Al, acc0, acc1)


def _head_n_body(h_ref, wp_ref, bp_ref, t_ref):
    h = h_ref[...]
    wp = wp_ref[...]
    s1 = jnp.dot(h, wp[:H, :], preferred_element_type=jnp.float32) + bp_ref[...]
    s2 = jnp.dot(h, wp[H:2 * H, :], preferred_element_type=jnp.float32)
    z = jnp.zeros((h.shape[0], H - 2), jnp.float32)
    t_ref[...] = jnp.concatenate([s1, s2, z], axis=1)


def _head_n(h, Wp, bp):
    bs = 2000
    t = pl.pallas_call(
        _head_n_body,
        grid=(N_NODES // bs,),
        in_specs=[
            pl.BlockSpec((bs, H), lambda i: (i, 0)),
            pl.BlockSpec((3 * H, 1), lambda i: (0, 0)),
            pl.BlockSpec((1, 1), lambda i: (0, 0)),
        ],
        out_specs=pl.BlockSpec((bs, H), lambda i: (i, 0)),
        out_shape=_f32(N_NODES, H),
    )(h, Wp, bp.reshape(1, 1))
    return t


def _head_e_body(lo_ref, hi_ref, wp_ref, t_ref):
    i = pl.program_id(0)
    wp = wp_ref[...]
    s3 = (jnp.dot(lo_ref[...], wp[2 * H:2 * H + HH, :],
                  preferred_element_type=jnp.float32)
          + jnp.dot(hi_ref[...], wp[2 * H + HH:, :],
                    preferred_element_type=jnp.float32))
    rows = s3.shape[0] // 128
    t_ref[pl.ds(i * rows, rows), :] = s3.reshape(rows, 128)


def _head_e(ef0, ef1, Wp):
    bs = 6400
    return pl.pallas_call(
        _head_e_body,
        grid=(N_EDGES // bs,),
        in_specs=[
            pl.BlockSpec((bs, HH), lambda i: (i, 0)),
            pl.BlockSpec((bs, HH), lambda i: (i, 0)),
            pl.BlockSpec((3 * H, 1), lambda i: (0, 0)),
        ],
        out_specs=pl.BlockSpec((ER, 128), lambda i: (0, 0)),
        out_shape=_f32(ER, 128),
    )(ef0, ef1, Wp)


def _edge_core(s, co, src_h, dst_h, efc_h, ef_h, db_h, he_h, efo_h,
               src_v, dst_v, efc_v, ef_v, db_v, he_v, ms_v, acc_s, sem):
    def blk(b, _):
        base = s * EPT + b * NB
        pltpu.sync_copy(src_h.at[pl.ds(base, NB)], src_v)
        pltpu.sync_copy(dst_h.at[pl.ds(base, NB)], dst_v)
        cp1 = pltpu.async_copy(db_h.at[src_v], db_v, sem)
        cp2 = pltpu.async_copy(he_h.at[dst_v], he_v, sem)
        pltpu.sync_copy(efc_h.at[pl.ds(base, NB)], efc_v)
        pltpu.sync_copy(ef_h.at[pl.ds(base, NB)], ef_v)
        cp1.wait()
        cp2.wait()

        def row(i, _):
            for j in range(HH // 16):
                sl = pl.ds(j * 16, 16)
                eh = efc_v[i, sl] + db_v[i, pl.ds(j * 16, 16)] \
                    + he_v[i, pl.ds(co + j * 16, 16)]
                sg = 1.0 / (1.0 + jnp.exp(-eh))
                ef_v[i, sl] = ef_v[i, sl] + jnp.maximum(eh, 0.0)
                ms_v[i, sl] = sg * db_v[i, pl.ds(HH + j * 16, 16)]
                ms_v[i, pl.ds(HH + j * 16, 16)] = sg
            return 0

        lax.fori_loop(0, NB, row, 0)
        pltpu.sync_copy(ef_v, efo_h.at[pl.ds(base, NB)])
        pltpu.sync_copy(ms_v, acc_s.at[dst_v], add=True)
        return 0

    lax.fori_loop(0, NBLK, blk, 0)


def _edge_sc_body(src_h, dst_h, efc0, efc1, ef0, ef1, db0, db1, he_h,
                  efo0, efo1, acc0, acc1,
                  src_v, dst_v, efc_v, ef_v, db_v, he_v, ms_v, zero_v,
                  acc_s, sem):
    c = lax.axis_index("c")
    s = lax.axis_index("s")

    def zrow(i, _):
        for j in range(H // 16):
            zero_v[i, pl.ds(j * 16, 16)] = jnp.zeros((16,), jnp.float32)
        return 0

    lax.fori_loop(0, ZB, zrow, 0)
    for k in range(NZB):
        r0 = s * NPT + k * ZB
        pltpu.sync_copy(zero_v, acc_s.at[pl.ds(r0, ZB)])

    @pl.when(s == NS - 1)
    def _():
        rt = NPT * NS
        pltpu.sync_copy(zero_v.at[pl.ds(0, NTAIL)], acc_s.at[pl.ds(rt, NTAIL)])

    plsc.subcore_barrier()

    @pl.when(c == 0)
    def _():
        _edge_core(s, 0, src_h, dst_h, efc0, ef0, db0, he_h, efo0,
                   src_v, dst_v, efc_v, ef_v, db_v, he_v, ms_v, acc_s, sem)

    @pl.when(c == 1)
    def _():
        _edge_core(s, HH, src_h, dst_h, efc1, ef1, db1, he_h, efo1,
                   src_v, dst_v, efc_v, ef_v, db_v, he_v, ms_v, acc_s, sem)

    plsc.subcore_barrier()

    ranges = [(s * NPT + k * ZB, ZB) for k in range(NZB)]

    @pl.when(c == 0)
    def _():
        for r0, rn in ranges:
            pltpu.sync_copy(acc_s.at[pl.ds(r0, rn)], acc0.at[pl.ds(r0, rn)])

        @pl.when(s == NS - 1)
        def _():
            rt = NPT * NS
            pltpu.sync_copy(acc_s.at[pl.ds(rt, NTAIL)],
                            acc0.at[pl.ds(rt, NTAIL)])

    @pl.when(c == 1)
    def _():
        for r0, rn in ranges:
            pltpu.sync_copy(acc_s.at[pl.ds(r0, rn)], acc1.at[pl.ds(r0, rn)])

        @pl.when(s == NS - 1)
        def _():
            rt = NPT * NS
            pltpu.sync_copy(acc_s.at[pl.ds(rt, NTAIL)],
                            acc1.at[pl.ds(rt, NTAIL)])


_edge_sc = functools.partial(
    pl.kernel,
    out_type=[_f32(N_EDGES, HH)] * 2 + [_f32(N_NODES, H)] * 2,
    mesh=_mesh,
    scratch_types=[
        pltpu.VMEM((NB,), jnp.int32),
        pltpu.VMEM((NB,), jnp.int32),
        pltpu.VMEM((NB, HH), jnp.float32),
        pltpu.VMEM((NB, HH), jnp.float32),
        pltpu.VMEM((NB, H), jnp.float32),
        pltpu.VMEM((NB, H), jnp.float32),
        pltpu.VMEM((NB, H), jnp.float32),
        pltpu.VMEM((ZB, H), jnp.float32),
        pltpu.VMEM_SHARED((N_NODES, H), jnp.float32),
        pltpu.SemaphoreType.DMA,
    ],
)(_edge_sc_body)


CB = 128
PPT = P // (2 * NS)


def _head_sc_body(sub_h, tsrc_h, tdst_h, t12_h, t3_h, out_h,
                  sid_v, rid_v, lane_v, u_v, v_v, eri_v, er_v, nr1_v, nr2_v,
                  o_v, sem):
    c = lax.axis_index("c")
    s = lax.axis_index("s")
    wid = c * NS + s

    def blk(k, _):
        base = wid * PPT + k * CB
        pltpu.sync_copy(sub_h.at[pl.ds(base, CB)], sid_v)
        for j in range(CB // 16):
            sl = pl.ds(j * 16, 16)
            sid = sid_v[sl]
            rid_v[sl] = lax.shift_right_logical(sid, 7)
            lane_v[sl] = lax.bitwise_and(sid, 127)
        g1 = pltpu.async_copy(tsrc_h.at[rid_v], eri_v, sem)
        g1.wait()
        for j in range(CB // 16):
            sl = pl.ds(j * 16, 16)
            i0 = lax.iota(jnp.int32, 16) + j * 16
            u_v[sl] = plsc.load_gather(eri_v, [i0, lane_v[sl]])
        g2 = pltpu.async_copy(tdst_h.at[rid_v], eri_v, sem)
        g2.wait()
        for j in range(CB // 16):
            sl = pl.ds(j * 16, 16)
            i0 = lax.iota(jnp.int32, 16) + j * 16
            v_v[sl] = plsc.load_gather(eri_v, [i0, lane_v[sl]])
        g3 = pltpu.async_copy(t12_h.at[u_v], nr1_v, sem)
        g4 = pltpu.async_copy(t12_h.at[v_v], nr2_v, sem)
        g5 = pltpu.async_copy(t3_h.at[rid_v], er_v, sem)
        g3.wait()
        g4.wait()
        g5.wait()
        for j in range(CB // 16):
            sl = pl.ds(j * 16, 16)
            i0 = lax.iota(jnp.int32, 16) + j * 16
            z = jnp.zeros((16,), jnp.int32)
            s1 = plsc.load_gather(nr1_v, [i0, z])
            s2 = plsc.load_gather(nr2_v, [i0, z + 1])
            s3 = plsc.load_gather(er_v, [i0, lane_v[sl]])
            o_v[sl] = s1 + s2 + s3
        pltpu.sync_copy(o_v, out_h.at[pl.ds(base, CB)])
        return 0

    lax.fori_loop(0, PPT // CB, blk, 0)


_head_sc = functools.partial(
    pl.kernel,
    out_type=_f32(P),
    mesh=_mesh,
    compiler_params=pltpu.CompilerParams(needs_layout_passes=False),
    scratch_types=[
        pltpu.VMEM((CB,), jnp.int32),
        pltpu.VMEM((CB,), jnp.int32),
        pltpu.VMEM((CB,), jnp.int32),
        pltpu.VMEM((CB,), jnp.int32),
        pltpu.VMEM((CB,), jnp.int32),
        pltpu.VMEM((CB, 128), jnp.int32),
        pltpu.VMEM((CB, 128), jnp.float32),
        pltpu.VMEM((CB, 128), jnp.float32),
        pltpu.VMEM((CB, 128), jnp.float32),
        pltpu.VMEM((CB,), jnp.float32),
        pltpu.SemaphoreType.DMA,
    ],
)(_head_sc_body)


def kernel(x, e, edge_index, sub_edge_ids, Wn, bn, We, be, A, B, C, D, Ee,
           Wp, bp):
    src = edge_index[0]
    dst = edge_index[1]
    h = _enc_h(x, Wn, bn)
    ef0, ef1 = _enc_e(e, We, be)
    for l in range(L):
        db0, db1, he = _proj(h, D[l], Ee[l], B[l])
        efc0, efc1 = _efc(ef0, ef1, C[l])
        ef0, ef1, acc0, acc1 = _edge_sc(
            src, dst, efc0, efc1, ef0, ef1, db0, db1, he)
        h = _update(h, A[l], acc0, acc1)
    t12 = _head_n(h, Wp, bp)
    t3 = _head_e(ef0, ef1, Wp)
    tsrc = src.reshape(ER, 128)
    tdst = dst.reshape(ER, 128)
    scores = _head_sc(sub_edge_ids, tsrc, tdst, t12, t3)
    return scores.reshape(P, 1)


# speedup vs baseline: 2.6341x; 1.6814x over previous
import functools

import jax
import jax.numpy as jnp
from jax import lax
from jax.experimental import pallas as pl
from jax.experimental.pallas import tpu as pltpu
from jax.experimental.pallas import tpu_sc as plsc

N_NODES = 10000
N_EDGES = 160000
D_NODE = 128
D_EDGE = 16
H = 128
HH = 64
L = 3
P = 32768

NS = 16
EPT = N_EDGES // NS
NB = 40
NBLK = EPT // NB
NPT = 624
ZB = 104
NZB = NPT // ZB
NTAIL = N_NODES - NPT * NS

ER = N_EDGES // 128
NR = 80
NPAD = NR * 128

_mesh = plsc.VectorSubcoreMesh(core_axis_name="c", subcore_axis_name="s")


def _f32(*shape):
    return jax.ShapeDtypeStruct(shape, jnp.float32)


def _enc_h_body(x_ref, wn_ref, bn_ref, h_ref):
    h_ref[...] = jnp.dot(x_ref[...], wn_ref[...],
                         preferred_element_type=jnp.float32) + bn_ref[...]


def _enc_h(x, Wn, bn):
    bs = 2000
    return pl.pallas_call(
        _enc_h_body,
        grid=(N_NODES // bs,),
        in_specs=[
            pl.BlockSpec((bs, D_NODE), lambda i: (i, 0)),
            pl.BlockSpec((D_NODE, H), lambda i: (0, 0)),
            pl.BlockSpec((1, H), lambda i: (0, 0)),
        ],
        out_specs=pl.BlockSpec((bs, H), lambda i: (i, 0)),
        out_shape=_f32(N_NODES, H),
    )(x, Wn, bn.reshape(1, H))


def _enc_e_body(e_ref, we_ref, be_ref, lo_ref, hi_ref):
    r = jnp.dot(e_ref[...], we_ref[...],
                preferred_element_type=jnp.float32) + be_ref[...]
    lo_ref[...] = r[:, :HH]
    hi_ref[...] = r[:, HH:]


def _enc_e(e, We, be):
    bs = 4000
    return pl.pallas_call(
        _enc_e_body,
        grid=(N_EDGES // bs,),
        in_specs=[
            pl.BlockSpec((bs, D_EDGE), lambda i: (i, 0)),
            pl.BlockSpec((D_EDGE, H), lambda i: (0, 0)),
            pl.BlockSpec((1, H), lambda i: (0, 0)),
        ],
        out_specs=[pl.BlockSpec((bs, HH), lambda i: (i, 0))] * 2,
        out_shape=[_f32(N_EDGES, HH)] * 2,
    )(e, We, be.reshape(1, H))


def _proj_body(h_ref, d_ref, ee_ref, b_ref, db0_ref, db1_ref, he_ref):
    h = h_ref[...]
    rd = jnp.dot(h, d_ref[...], preferred_element_type=jnp.float32)
    rb = jnp.dot(h, b_ref[...], preferred_element_type=jnp.float32)
    he_ref[...] = jnp.dot(h, ee_ref[...], preferred_element_type=jnp.float32)
    db0_ref[...] = jnp.concatenate([rd[:, :HH], rb[:, :HH]], axis=1)
    db1_ref[...] = jnp.concatenate([rd[:, HH:], rb[:, HH:]], axis=1)


def _proj(h, Dl, Eel, Bl):
    bs = 2000
    return pl.pallas_call(
        _proj_body,
        grid=(N_NODES // bs,),
        in_specs=[
            pl.BlockSpec((bs, H), lambda i: (i, 0)),
            pl.BlockSpec((H, H), lambda i: (0, 0)),
            pl.BlockSpec((H, H), lambda i: (0, 0)),
            pl.BlockSpec((H, H), lambda i: (0, 0)),
        ],
        out_specs=[pl.BlockSpec((bs, H), lambda i: (i, 0))] * 3,
        out_shape=[_f32(N_NODES, H)] * 3,
    )(h, Dl, Eel, Bl)


def _efc_body(lo_ref, hi_ref, c_ref, o0_ref, o1_ref):
    c = c_ref[...]
    r = (jnp.dot(lo_ref[...], c[:HH, :], preferred_element_type=jnp.float32)
         + jnp.dot(hi_ref[...], c[HH:, :], preferred_element_type=jnp.float32))
    o0_ref[...] = r[:, :HH]
    o1_ref[...] = r[:, HH:]


def _efc(ef0, ef1, Cl):
    bs = 4000
    return pl.pallas_call(
        _efc_body,
        grid=(N_EDGES // bs,),
        in_specs=[
            pl.BlockSpec((bs, HH), lambda i: (i, 0)),
            pl.BlockSpec((bs, HH), lambda i: (i, 0)),
            pl.BlockSpec((H, H), lambda i: (0, 0)),
        ],
        out_specs=[pl.BlockSpec((bs, HH), lambda i: (i, 0))] * 2,
        out_shape=[_f32(N_EDGES, HH)] * 2,
    )(ef0, ef1, Cl)


def _update_body(h_ref, a_ref, x0_ref, x1_ref, o_ref):
    h = h_ref[...]
    a0 = x0_ref[...]
    a1 = x1_ref[...]
    agg = jnp.concatenate([a0[:, :HH], a1[:, :HH]], axis=1)
    den = jnp.concatenate([a0[:, HH:], a1[:, HH:]], axis=1) + 1e-6
    hn = jnp.dot(h, a_ref[...], preferred_element_type=jnp.float32) + agg / den
    o_ref[...] = h + jnp.maximum(hn, 0.0)


def _update(h, Al, acc0, acc1):
    bs = 2000
    return pl.pallas_call(
        _update_body,
        grid=(N_NODES // bs,),
        in_specs=[
            pl.BlockSpec((bs, H), lambda i: (i, 0)),
            pl.BlockSpec((H, H), lambda i: (0, 0)),
            pl.BlockSpec((bs, H), lambda i: (i, 0)),
            pl.BlockSpec((bs, H), lambda i: (i, 0)),
        ],
        out_specs=pl.BlockSpec((bs, H), lambda i: (i, 0)),
        out_shape=_f32(N_NODES, H),
    )(h, Al, acc0, acc1)


def _head_n_body(h_ref, wp_ref, bp_ref, t_ref):
    h = h_ref[...]
    wp = wp_ref[...]
    s1 = jnp.dot(h, wp[:H, :], preferred_element_type=jnp.float32) + bp_ref[...]
    s2 = jnp.dot(h, wp[H:2 * H, :], preferred_element_type=jnp.float32)
    z = jnp.zeros((h.shape[0], H - 2), jnp.float32)
    t_ref[...] = jnp.concatenate([s1, s2, z], axis=1)


def _head_n(h, Wp, bp):
    bs = 2000
    t = pl.pallas_call(
        _head_n_body,
        grid=(N_NODES // bs,),
        in_specs=[
            pl.BlockSpec((bs, H), lambda i: (i, 0)),
            pl.BlockSpec((3 * H, 1), lambda i: (0, 0)),
            pl.BlockSpec((1, 1), lambda i: (0, 0)),
        ],
        out_specs=pl.BlockSpec((bs, H), lambda i: (i, 0)),
        out_shape=_f32(N_NODES, H),
    )(h, Wp, bp.reshape(1, 1))
    return t


def _head_e_body(lo_ref, hi_ref, wp_ref, t_ref):
    i = pl.program_id(0)
    wp = wp_ref[...]
    s3 = (jnp.dot(lo_ref[...], wp[2 * H:2 * H + HH, :],
                  preferred_element_type=jnp.float32)
          + jnp.dot(hi_ref[...], wp[2 * H + HH:, :],
                    preferred_element_type=jnp.float32))
    rows = s3.shape[0] // 128
    t_ref[pl.ds(i * rows, rows), :] = s3.reshape(rows, 128)


def _head_e(ef0, ef1, Wp):
    bs = 6400
    return pl.pallas_call(
        _head_e_body,
        grid=(N_EDGES // bs,),
        in_specs=[
            pl.BlockSpec((bs, HH), lambda i: (i, 0)),
            pl.BlockSpec((bs, HH), lambda i: (i, 0)),
            pl.BlockSpec((3 * H, 1), lambda i: (0, 0)),
        ],
        out_specs=pl.BlockSpec((ER, 128), lambda i: (0, 0)),
        out_shape=_f32(ER, 128),
    )(ef0, ef1, Wp)


def _edge_core(s, co, src_h, dst_h, efc_h, ef_h, db_h, he_h, efo_h,
               src_v, dst_v, efc_v, ef_v, db_v, he_v, ms_v, acc_s,
               sld, swr, ssc):
    def issue(q, b):
        base = s * EPT + b * NB
        pltpu.sync_copy(src_h.at[pl.ds(base, NB)], src_v[q])
        pltpu.sync_copy(dst_h.at[pl.ds(base, NB)], dst_v[q])
        pltpu.async_copy(efc_h.at[pl.ds(base, NB)], efc_v[q], sld[q])
        pltpu.async_copy(ef_h.at[pl.ds(base, NB)], ef_v[q], sld[q])
        pltpu.async_copy(db_h.at[src_v[q]], db_v[q], sld[q])
        pltpu.async_copy(he_h.at[dst_v[q]], he_v[q], sld[q])

    def wait_loads(q):
        b0 = s * EPT
        pltpu.make_async_copy(efc_h.at[pl.ds(b0, NB)], efc_v[q], sld[q]).wait()
        pltpu.make_async_copy(ef_h.at[pl.ds(b0, NB)], ef_v[q], sld[q]).wait()
        pltpu.make_async_copy(db_h.at[src_v[q]], db_v[q], sld[q]).wait()
        pltpu.make_async_copy(he_h.at[dst_v[q]], he_v[q], sld[q]).wait()

    def drain_write(q):
        pltpu.make_async_copy(ef_v[q], efo_h.at[pl.ds(s * EPT, NB)],
                              swr[q]).wait()

    def drain_scatter(q):
        pltpu.make_async_copy(ms_v, acc_s.at[dst_v[q]], ssc).wait()

    def compute(p):
        @plsc.parallel_loop(0, NB, unroll=2)
        def _(i):
            for j in range(HH // 16):
                sl = pl.ds(j * 16, 16)
                eh = efc_v[p][i, sl] + db_v[p][i, pl.ds(j * 16, 16)] \
                    + he_v[p][i, pl.ds(co + j * 16, 16)]
                sg = 1.0 / (1.0 + jnp.exp(-eh))
                ef_v[p][i, sl] = ef_v[p][i, sl] + jnp.maximum(eh, 0.0)
                ms_v[i, sl] = sg * db_v[p][i, pl.ds(HH + j * 16, 16)]
                ms_v[i, pl.ds(HH + j * 16, 16)] = sg

    def step(bi, p):
        b = 2 * bi + p
        q = 1 - p

        def pre():
            drain_scatter(q)
            drain_write(q)

        if p == 0:
            @pl.when(bi > 0)
            def _():
                pre()
        else:
            pre()

        if p == 1:
            @pl.when(bi < NBLK // 2 - 1)
            def _():
                issue(q, b + 1)
        else:
            issue(q, b + 1)

        wait_loads(p)
        compute(p)
        base = s * EPT + b * NB
        pltpu.async_copy(ef_v[p], efo_h.at[pl.ds(base, NB)], swr[p])
        pltpu.async_copy(ms_v, acc_s.at[dst_v[p]], ssc, add=True)

    issue(0, 0)

    def loop(bi, _):
        step(bi, 0)
        step(bi, 1)
        return 0

    lax.fori_loop(0, NBLK // 2, loop, 0)
    drain_scatter(1)
    drain_write(1)


def _edge_sc_body(src_h, dst_h, efc0, efc1, ef0, ef1, db0, db1, he_h,
                  efo0, efo1, acc0, acc1,
                  src_v0, src_v1, dst_v0, dst_v1, efc_v0, efc_v1,
                  ef_v0, ef_v1, db_v0, db_v1, he_v0, he_v1, ms_v,
                  acc_s, sld0, sld1, swr0, swr1, ssc):
    c = lax.axis_index("c")
    s = lax.axis_index("s")
    src_v = (src_v0, src_v1)
    dst_v = (dst_v0, dst_v1)
    efc_v = (efc_v0, efc_v1)
    ef_v = (ef_v0, ef_v1)
    db_v = (db_v0, db_v1)
    he_v = (he_v0, he_v1)
    sld = (sld0, sld1)
    swr = (swr0, swr1)

    def zrow(i, _):
        for j in range(H // 16):
            ms_v[i, pl.ds(j * 16, 16)] = jnp.zeros((16,), jnp.float32)
        return 0

    lax.fori_loop(0, NB, zrow, 0)
    for k in range(NPT // NB):
        r0 = s * NPT + k * NB
        pltpu.sync_copy(ms_v, acc_s.at[pl.ds(r0, NB)])
    rrem = NPT - (NPT // NB) * NB
    if rrem:
        pltpu.sync_copy(ms_v.at[pl.ds(0, rrem)],
                        acc_s.at[pl.ds(s * NPT + NPT - rrem, rrem)])

    @pl.when(s == NS - 1)
    def _():
        rt = NPT * NS
        pltpu.sync_copy(ms_v.at[pl.ds(0, NTAIL)], acc_s.at[pl.ds(rt, NTAIL)])

    plsc.subcore_barrier()

    @pl.when(c == 0)
    def _():
        _edge_core(s, 0, src_h, dst_h, efc0, ef0, db0, he_h, efo0,
                   src_v, dst_v, efc_v, ef_v, db_v, he_v, ms_v, acc_s,
                   sld, swr, ssc)

    @pl.when(c == 1)
    def _():
        _edge_core(s, HH, src_h, dst_h, efc1, ef1, db1, he_h, efo1,
                   src_v, dst_v, efc_v, ef_v, db_v, he_v, ms_v, acc_s,
                   sld, swr, ssc)

    plsc.subcore_barrier()

    ranges = [(s * NPT + k * ZB, ZB) for k in range(NZB)]

    @pl.when(c == 0)
    def _():
        for r0, rn in ranges:
            pltpu.sync_copy(acc_s.at[pl.ds(r0, rn)], acc0.at[pl.ds(r0, rn)])

        @pl.when(s == NS - 1)
        def _():
            rt = NPT * NS
            pltpu.sync_copy(acc_s.at[pl.ds(rt, NTAIL)],
                            acc0.at[pl.ds(rt, NTAIL)])

    @pl.when(c == 1)
    def _():
        for r0, rn in ranges:
            pltpu.sync_copy(acc_s.at[pl.ds(r0, rn)], acc1.at[pl.ds(r0, rn)])

        @pl.when(s == NS - 1)
        def _():
            rt = NPT * NS
            pltpu.sync_copy(acc_s.at[pl.ds(rt, NTAIL)],
                            acc1.at[pl.ds(rt, NTAIL)])


_edge_sc = functools.partial(
    pl.kernel,
    out_type=[_f32(N_EDGES, HH)] * 2 + [_f32(N_NODES, H)] * 2,
    mesh=_mesh,
    scratch_types=(
        [pltpu.VMEM((NB,), jnp.int32)] * 4
        + [pltpu.VMEM((NB, HH), jnp.float32)] * 4
        + [pltpu.VMEM((NB, H), jnp.float32)] * 5
        + [pltpu.VMEM_SHARED((N_NODES, H), jnp.float32)]
        + [pltpu.SemaphoreType.DMA] * 5
    ),
)(_edge_sc_body)


CB = 128
PPT = P // (2 * NS)


def _head_sc_body(sub_h, tsrc_h, tdst_h, t12_h, t3_h, out_h,
                  sid_v, rid_v, lane_v, u_v, v_v, eri_v, er_v, nr1_v, nr2_v,
                  o_v, sem):
    c = lax.axis_index("c")
    s = lax.axis_index("s")
    wid = c * NS + s

    def blk(k, _):
        base = wid * PPT + k * CB
        pltpu.sync_copy(sub_h.at[pl.ds(base, CB)], sid_v)
        for j in range(CB // 16):
            sl = pl.ds(j * 16, 16)
            sid = sid_v[sl]
            rid_v[sl] = lax.shift_right_logical(sid, 7)
            lane_v[sl] = lax.bitwise_and(sid, 127)
        g1 = pltpu.async_copy(tsrc_h.at[rid_v], eri_v, sem)
        g1.wait()
        for j in range(CB // 16):
            sl = pl.ds(j * 16, 16)
            i0 = lax.iota(jnp.int32, 16) + j * 16
            u_v[sl] = plsc.load_gather(eri_v, [i0, lane_v[sl]])
        g2 = pltpu.async_copy(tdst_h.at[rid_v], eri_v, sem)
        g2.wait()
        for j in range(CB // 16):
            sl = pl.ds(j * 16, 16)
            i0 = lax.iota(jnp.int32, 16) + j * 16
            v_v[sl] = plsc.load_gather(eri_v, [i0, lane_v[sl]])
        g3 = pltpu.async_copy(t12_h.at[u_v], nr1_v, sem)
        g4 = pltpu.async_copy(t12_h.at[v_v], nr2_v, sem)
        g5 = pltpu.async_copy(t3_h.at[rid_v], er_v, sem)
        g3.wait()
        g4.wait()
        g5.wait()
        for j in range(CB // 16):
            sl = pl.ds(j * 16, 16)
            i0 = lax.iota(jnp.int32, 16) + j * 16
            z = jnp.zeros((16,), jnp.int32)
            s1 = plsc.load_gather(nr1_v, [i0, z])
            s2 = plsc.load_gather(nr2_v, [i0, z + 1])
            s3 = plsc.load_gather(er_v, [i0, lane_v[sl]])
            o_v[sl] = s1 + s2 + s3
        pltpu.sync_copy(o_v, out_h.at[pl.ds(base, CB)])
        return 0

    lax.fori_loop(0, PPT // CB, blk, 0)


_head_sc = functools.partial(
    pl.kernel,
    out_type=_f32(P),
    mesh=_mesh,
    compiler_params=pltpu.CompilerParams(needs_layout_passes=False),
    scratch_types=[
        pltpu.VMEM((CB,), jnp.int32),
        pltpu.VMEM((CB,), jnp.int32),
        pltpu.VMEM((CB,), jnp.int32),
        pltpu.VMEM((CB,), jnp.int32),
        pltpu.VMEM((CB,), jnp.int32),
        pltpu.VMEM((CB, 128), jnp.int32),
        pltpu.VMEM((CB, 128), jnp.float32),
        pltpu.VMEM((CB, 128), jnp.float32),
        pltpu.VMEM((CB, 128), jnp.float32),
        pltpu.VMEM((CB,), jnp.float32),
        pltpu.SemaphoreType.DMA,
    ],
)(_head_sc_body)


def kernel(x, e, edge_index, sub_edge_ids, Wn, bn, We, be, A, B, C, D, Ee,
           Wp, bp):
    src = edge_index[0]
    dst = edge_index[1]
    h = _enc_h(x, Wn, bn)
    ef0, ef1 = _enc_e(e, We, be)
    for l in range(L):
        db0, db1, he = _proj(h, D[l], Ee[l], B[l])
        efc0, efc1 = _efc(ef0, ef1, C[l])
        ef0, ef1, acc0, acc1 = _edge_sc(
            src, dst, efc0, efc1, ef0, ef1, db0, db1, he)
        h = _update(h, A[l], acc0, acc1)
    t12 = _head_n(h, Wp, bp)
    t3 = _head_e(ef0, ef1, Wp)
    tsrc = src.reshape(ER, 128)
    tdst = dst.reshape(ER, 128)
    scores = _head_sc(sub_edge_ids, tsrc, tdst, t12, t3)
    return scores.reshape(P, 1)
